# Initial kernel scaffold; baseline (speedup 1.0000x reference)
import jax
import jax.numpy as jnp
from jax.experimental import pallas as pl


def kernel(head, rel, timestamp, target, E_x, E_y, E_z, cores, R_x, R_y, R_z, time_mat, bn_w, bn_b):
    raise NotImplementedError("write your pallas kernel here")


# speedup vs baseline: 1.3300x; 1.3300x over previous
import functools

import jax
import jax.numpy as jnp
from jax import lax
from jax.experimental import pallas as pl
from jax.experimental.pallas import tpu as pltpu
from jax.experimental.pallas import tpu_sc as plsc

E_NUM = 100000
R_NUM = 500
T_NUM = 365
D = 128
B = 4096

_NC = 2
_NS = 16
_NW = _NC * _NS
_BPW = B // _NW


def _sc_gather_body(ex, ey, ez, co, rxt, ryt, rzt, head, target, rel, hcidx,
                    hx, hy, hz, ext, eyt, ezt, rx, ry, rz, cc,
                    idx1_v, idx2_v, rows1_v, rows2_v, sem):
    wid = lax.axis_index("s") * _NC + lax.axis_index("c")
    base = wid * _BPW

    def gather128(tbl, idx_hbm, out_hbm):
        pltpu.sync_copy(idx_hbm.at[pl.ds(base, _BPW)], idx1_v)
        pltpu.async_copy(tbl.at[idx1_v], rows1_v, sem).wait()
        pltpu.sync_copy(rows1_v, out_hbm.at[pl.ds(base, _BPW)])

    pltpu.sync_copy(hcidx.at[pl.ds(2 * base, 2 * _BPW)], idx2_v)
    pltpu.async_copy(co.at[idx2_v], rows2_v, sem).wait()
    pltpu.sync_copy(rows2_v, cc.at[pl.ds(2 * base, 2 * _BPW)])

    gather128(ex, head, hx)
    gather128(ey, head, hy)
    gather128(ez, head, hz)
    gather128(ex, target, ext)
    gather128(ey, target, eyt)
    gather128(ez, target, ezt)
    gather128(rxt, rel, rx)
    gather128(ryt, rel, ry)
    gather128(rzt, rel, rz)


def _sc_gather(E_x, E_y, E_z, cores, R_x, R_y, R_z, head, target, rel, hcidx):
    f32 = jnp.float32
    out_type = (
        [jax.ShapeDtypeStruct((B, D), f32)] * 9
        + [jax.ShapeDtypeStruct((2 * B, D), f32)]
    )
    mesh = plsc.VectorSubcoreMesh(core_axis_name="c", subcore_axis_name="s")
    fn = pl.kernel(
        _sc_gather_body,
        out_type=out_type,
        mesh=mesh,
        scratch_types=[
            pltpu.VMEM((_BPW,), jnp.int32),
            pltpu.VMEM((2 * _BPW,), jnp.int32),
            pltpu.VMEM((_BPW, D), f32),
            pltpu.VMEM((2 * _BPW, D), f32),
            pltpu.SemaphoreType.DMA,
        ],
    )
    return fn(E_x, E_y, E_z, cores, R_x, R_y, R_z, head, target, rel, hcidx)


_UNROLL = 4


def _mv_body(ts_ref, time_ref, vecs_ref, out1_ref, out2_ref):
    def body(i, carry):
        for u in range(_UNROLL):
            k = i * _UNROLL + u
            t = ts_ref[k]
            m = time_ref[t]
            v = vecs_ref[pl.ds(2 * k, 2), :]
            o = lax.dot_general(v, m, (((1,), (1,)), ((), ())),
                                preferred_element_type=jnp.float32)
            out1_ref[pl.ds(k, 1), :] = o[0:1]
            out2_ref[pl.ds(k, 1), :] = o[1:2]
        return carry

    lax.fori_loop(0, B // _UNROLL, body, 0)


def _mv_call(timestamp, time_mat, cc, interpret=False):
    f32 = jnp.float32
    return pl.pallas_call(
        _mv_body,
        out_shape=[jax.ShapeDtypeStruct((B, D), f32)] * 2,
        in_specs=[
            pl.BlockSpec(memory_space=pltpu.SMEM),
            pl.BlockSpec(memory_space=pltpu.VMEM),
            pl.BlockSpec(memory_space=pltpu.VMEM),
        ],
        out_specs=[pl.BlockSpec(memory_space=pltpu.VMEM)] * 2,
        interpret=interpret,
    )(timestamp, time_mat, cc)


def _fuse_body(hct_ref, ctm_ref, hx_ref, hy_ref, hz_ref,
               ext_ref, eyt_ref, ezt_ref, rx_ref, ry_ref, rz_ref,
               w_ref, b_ref, sx_ref, sy_ref, sz_ref, sct_ref):
    w = w_ref[:]
    b = b_ref[:]

    def bn(x):
        mean = jnp.mean(x, axis=0, keepdims=True)
        var = jnp.mean((x - mean) ** 2, axis=0, keepdims=True)
        return (x - mean) / jnp.sqrt(var + 1e-5) * w + b

    h_ct = bn(hct_ref[:])
    h_x = bn(hx_ref[:])
    h_y = bn(hy_ref[:])
    h_z = bn(hz_ref[:])

    r_x = rx_ref[:]
    r_y = ry_ref[:]
    r_z = rz_ref[:]
    length = jnp.sqrt(r_x * r_x + r_y * r_y + r_z * r_z)
    r_v_rate = jax.nn.sigmoid(length)
    inv_len = 1.0 / length
    r_x = r_x * inv_len
    r_y = r_y * inv_len
    r_z = r_z * inv_len
    gamma = 1.0 / jnp.sqrt(1.0 - r_v_rate * r_v_rate)
    gm1 = gamma - 1.0
    grv = gamma * r_v_rate

    t_ct = gamma * h_ct + grv * (r_x * h_x + r_y * h_y + r_z * h_z)
    t_x = (r_x * grv) * h_ct + (1.0 + r_x * r_x * gm1) * h_x \
        + (r_x * r_y * gm1) * h_y + (r_x * r_z * gm1) * h_z
    t_y = (r_y * grv) * h_ct + (r_x * r_y * gm1) * h_x \
        + (1.0 + r_y * r_y * gm1) * h_y + (r_z * r_y * gm1) * h_z
    t_z = (r_z * grv) * h_ct + (r_x * r_z * gm1) * h_x \
        + (r_y * r_z * gm1) * h_y + (1.0 + r_z * r_z * gm1) * h_z

    sx_ref[:] = jnp.sum(t_x * ext_ref[:], axis=1)
    sy_ref[:] = jnp.sum(t_y * eyt_ref[:], axis=1)
    sz_ref[:] = jnp.sum(t_z * ezt_ref[:], axis=1)
    sct_ref[:] = jnp.sum(t_ct * ctm_ref[:], axis=1)


def _fuse_call(hct, ctm, hx, hy, hz, ext, eyt, ezt, rx, ry, rz, bn_w, bn_b,
               interpret=False):
    f32 = jnp.float32
    return pl.pallas_call(
        _fuse_body,
        out_shape=[jax.ShapeDtypeStruct((B,), f32)] * 4,
        interpret=interpret,
    )(hct, ctm, hx, hy, hz, ext, eyt, ezt, rx, ry, rz, bn_w, bn_b)


def kernel(head, rel, timestamp, target, E_x, E_y, E_z, cores,
           R_x, R_y, R_z, time_mat, bn_w, bn_b):
    head = head.astype(jnp.int32)
    rel = rel.astype(jnp.int32)
    timestamp = timestamp.astype(jnp.int32)
    target = target.astype(jnp.int32)

    hcidx = jnp.stack([head, target], axis=1).reshape(-1)

    (hx, hy, hz, ext, eyt, ezt, rx, ry, rz, cc) = _sc_gather(
        E_x, E_y, E_z, cores, R_x, R_y, R_z, head, target, rel, hcidx)

    hct, ctm = _mv_call(timestamp, time_mat, cc)

    sx, sy, sz, sct = _fuse_call(hct, ctm, hx, hy, hz, ext, eyt, ezt,
                                 rx, ry, rz, bn_w, bn_b)
    return sx, sy, sz, sct
